# baseline (speedup 1.0000x reference)
import functools

import numpy as np
import jax
import jax.numpy as jnp
from jax import lax
from jax.experimental import pallas as pl
from jax.experimental.pallas import tpu as pltpu, tpu_sc as plsc

B, T, D = 128, 4096, 64
NC, NS = 2, 16
NW = NC * NS
ROWS = B * T
RPW = ROWS // NW
G = 128
KFIRE = 8
CHUNK = G * KFIRE
NIT = RPW // CHUNK


def _build_flat_idx() -> np.ndarray:
    rng = np.random.RandomState(0)
    orig = np.arange(T)
    num_segs = rng.randint(1, 5, size=B)
    warp = np.tile(orig, (B, 1))
    for i in range(B):
        if num_segs[i] > 1:
            splits = np.array_split(orig, num_segs[i])
            perm = rng.permutation(len(splits))
            warp[i] = np.concatenate([splits[j] for j in perm]).ravel()
    flat = warp.astype(np.int64) + np.arange(B, dtype=np.int64)[:, None] * T
    return flat.reshape(NW, RPW // G, G).astype(np.int32)


_FLAT_IDX = _build_flat_idx()


@functools.partial(
    pl.kernel,
    out_type=jax.ShapeDtypeStruct((ROWS, D), jnp.float32),
    mesh=plsc.VectorSubcoreMesh(core_axis_name="c", subcore_axis_name="s"),
    scratch_types=[
        pltpu.VMEM((RPW // G, G), jnp.int32),
        pltpu.VMEM((CHUNK, D), jnp.float32),
        pltpu.SemaphoreType.DMA,
    ],
    compiler_params=pltpu.CompilerParams(use_tc_tiling_on_sc=False),
)
def _sc_gather(table_hbm, idx_hbm, out_hbm, idx_v, rows_v, sem):
    wid = lax.axis_index("s") * NC + lax.axis_index("c")
    base = wid * RPW
    pltpu.sync_copy(idx_hbm.at[wid], idx_v)

    def body(it, carry):
        copies = []
        for g in range(KFIRE):
            j = it * KFIRE + g
            copies.append(pltpu.async_copy(
                table_hbm.at[idx_v.at[j]],
                rows_v.at[pl.ds(g * G, G)],
                sem,
            ))
        for c in copies:
            c.wait()
        pltpu.sync_copy(rows_v, out_hbm.at[pl.ds(base + it * CHUNK, CHUNK)])
        return carry

    lax.fori_loop(0, NIT, body, 0)


def kernel(x):
    table = x.reshape(ROWS, D)
    idx = jnp.asarray(_FLAT_IDX)
    out = _sc_gather(table, idx)
    return out.reshape(B, T, D)


# speedup vs baseline: 1.0054x; 1.0054x over previous
import functools

import numpy as np
import jax
import jax.numpy as jnp
from jax import lax
from jax.experimental import pallas as pl
from jax.experimental.pallas import tpu as pltpu, tpu_sc as plsc

B, T, D = 128, 4096, 64
NC, NS = 2, 16
NW = NC * NS
NELEM = B * T * D
C = 512
CE = C * D


def _build_plan():
    rng = np.random.RandomState(0)
    orig = np.arange(T)
    num_segs = rng.randint(1, 5, size=B)
    warp = np.tile(orig, (B, 1))
    for i in range(B):
        if num_segs[i] > 1:
            splits = np.array_split(orig, num_segs[i])
            perm = rng.permutation(len(splits))
            warp[i] = np.concatenate([splits[j] for j in perm]).ravel()

    chunks = []
    for i in range(B):
        row = warp[i]
        breaks = np.flatnonzero(np.diff(row) != 1)
        starts = np.concatenate(([0], breaks + 1))
        ends = np.concatenate((breaks + 1, [T]))
        for s, e in zip(starts, ends):
            seg_len = e - s
            src0, dst0 = int(row[s]), int(s)
            n = -(-seg_len // C)
            for c in range(n):
                off = min(c * C, seg_len - C)
                chunks.append(((i * T + src0 + off) * D,
                               (i * T + dst0 + off) * D))

    while len(chunks) % NW:
        chunks.append(chunks[len(chunks) % NW])
    nchw = len(chunks) // NW
    ngrp = -(-nchw // 8)
    ncol = ngrp * 8 + 16
    src = np.zeros((NW, ncol), dtype=np.int32)
    dst = np.zeros((NW, ncol), dtype=np.int32)
    for k, (s, d) in enumerate(chunks):
        src[k % NW, k // NW] = s
        dst[k % NW, k // NW] = d
    return src, dst, nchw, ngrp, ncol


_PLAN_SRC, _PLAN_DST, NCHW, NGRP, NCOL = _build_plan()


@functools.partial(
    pl.kernel,
    out_type=jax.ShapeDtypeStruct((NELEM,), jnp.float32),
    mesh=plsc.VectorSubcoreMesh(core_axis_name="c", subcore_axis_name="s"),
    scratch_types=[
        pltpu.VMEM((NCOL,), jnp.int32),
        pltpu.VMEM((NCOL,), jnp.int32),
        pltpu.VMEM((CE,), jnp.float32),
        pltpu.VMEM((CE,), jnp.float32),
        pltpu.SemaphoreType.DMA,
        pltpu.SemaphoreType.DMA,
        pltpu.SemaphoreType.DMA,
    ],
)
def _sc_permute(x_hbm, src_hbm, dst_hbm, out_hbm,
                src_v, dst_v, buf0, buf1, rsem, wsem0, wsem1):
    wid = lax.axis_index("s") * NC + lax.axis_index("c")
    pltpu.sync_copy(src_hbm.at[wid], src_v)
    pltpu.sync_copy(dst_hbm.at[wid], dst_v)
    bufs = (buf0, buf1)
    wsems = (wsem0, wsem1)

    def body(g, carry):
        base = pl.multiple_of(g * 8, 8)
        svec = src_v[pl.ds(base, 16)]
        dvec = dst_v[pl.ds(base, 16)]
        for k in range(8):
            jj = g * 8 + k
            buf, wsem = bufs[k & 1], wsems[k & 1]

            @pl.when((jj >= 2) & (jj < NCHW))
            def _():
                pltpu.make_async_copy(
                    buf, out_hbm.at[pl.ds(0, CE)], wsem).wait()

            @pl.when(jj < NCHW)
            def _():
                s = pl.multiple_of(svec[k], D)
                d = pl.multiple_of(dvec[k], D)
                pltpu.async_copy(x_hbm.at[pl.ds(s, CE)], buf, rsem).wait()
                pltpu.async_copy(buf, out_hbm.at[pl.ds(d, CE)], wsem)

        return carry

    lax.fori_loop(0, NGRP, body, 0)
    pltpu.make_async_copy(buf0, out_hbm.at[pl.ds(0, CE)], wsem0).wait()
    pltpu.make_async_copy(buf1, out_hbm.at[pl.ds(0, CE)], wsem1).wait()


def kernel(x):
    out = _sc_permute(x.reshape(NELEM),
                      jnp.asarray(_PLAN_SRC), jnp.asarray(_PLAN_DST))
    return out.reshape(B, T, D)


# speedup vs baseline: 1.0058x; 1.0003x over previous
import functools

import numpy as np
import jax
import jax.numpy as jnp
from jax import lax
from jax.experimental import pallas as pl
from jax.experimental.pallas import tpu as pltpu, tpu_sc as plsc

B, T, D = 128, 4096, 64
NC, NS = 2, 16
NW = NC * NS
NELEM = B * T * D
C = 512
CE = C * D


def _build_plan():
    rng = np.random.RandomState(0)
    orig = np.arange(T)
    num_segs = rng.randint(1, 5, size=B)
    warp = np.tile(orig, (B, 1))
    for i in range(B):
        if num_segs[i] > 1:
            splits = np.array_split(orig, num_segs[i])
            perm = rng.permutation(len(splits))
            warp[i] = np.concatenate([splits[j] for j in perm]).ravel()

    chunks = []
    for i in range(B):
        row = warp[i]
        breaks = np.flatnonzero(np.diff(row) != 1)
        starts = np.concatenate(([0], breaks + 1))
        ends = np.concatenate((breaks + 1, [T]))
        for s, e in zip(starts, ends):
            seg_len = e - s
            src0, dst0 = int(row[s]), int(s)
            n = -(-seg_len // C)
            for c in range(n):
                off = min(c * C, seg_len - C)
                chunks.append(((i * T + src0 + off) * D,
                               (i * T + dst0 + off) * D))

    while len(chunks) % NW:
        chunks.append(chunks[len(chunks) % NW])
    nchw = len(chunks) // NW
    ngrp = -(-nchw // 8)
    ncol = ngrp * 8 + 16
    src = np.zeros((NW, ncol), dtype=np.int32)
    dst = np.zeros((NW, ncol), dtype=np.int32)
    for k, (s, d) in enumerate(chunks):
        src[k % NW, k // NW] = s
        dst[k % NW, k // NW] = d
    return src, dst, nchw, ngrp, ncol


_PLAN_SRC, _PLAN_DST, NCHW, NGRP, NCOL = _build_plan()


@functools.partial(
    pl.kernel,
    out_type=jax.ShapeDtypeStruct((NELEM,), jnp.float32),
    mesh=plsc.VectorSubcoreMesh(core_axis_name="c", subcore_axis_name="s"),
    scratch_types=[
        pltpu.VMEM((NCOL,), jnp.int32),
        pltpu.VMEM((NCOL,), jnp.int32),
        pltpu.VMEM((CE,), jnp.float32),
        pltpu.VMEM((CE,), jnp.float32),
        pltpu.SemaphoreType.DMA,
        pltpu.SemaphoreType.DMA,
        pltpu.SemaphoreType.DMA,
    ],
    compiler_params=pltpu.CompilerParams(skip_device_barrier=True),
)
def _sc_permute(x_hbm, src_hbm, dst_hbm, out_hbm,
                src_v, dst_v, buf0, buf1, rsem, wsem0, wsem1):
    wid = lax.axis_index("s") * NC + lax.axis_index("c")
    pltpu.sync_copy(src_hbm.at[wid], src_v)
    pltpu.sync_copy(dst_hbm.at[wid], dst_v)
    bufs = (buf0, buf1)
    wsems = (wsem0, wsem1)

    def body(g, carry):
        base = pl.multiple_of(g * 8, 8)
        svec = src_v[pl.ds(base, 16)]
        dvec = dst_v[pl.ds(base, 16)]
        for k in range(8):
            jj = g * 8 + k
            buf, wsem = bufs[k & 1], wsems[k & 1]

            @pl.when((jj >= 2) & (jj < NCHW))
            def _():
                pltpu.make_async_copy(
                    buf, out_hbm.at[pl.ds(0, CE)], wsem).wait()

            @pl.when(jj < NCHW)
            def _():
                s = pl.multiple_of(svec[k], D)
                d = pl.multiple_of(dvec[k], D)
                pltpu.async_copy(x_hbm.at[pl.ds(s, CE)], buf, rsem).wait()
                pltpu.async_copy(buf, out_hbm.at[pl.ds(d, CE)], wsem)

        return carry

    lax.fori_loop(0, NGRP, body, 0)
    pltpu.make_async_copy(buf0, out_hbm.at[pl.ds(0, CE)], wsem0).wait()
    pltpu.make_async_copy(buf1, out_hbm.at[pl.ds(0, CE)], wsem1).wait()


def kernel(x):
    out = _sc_permute(x.reshape(NELEM),
                      jnp.asarray(_PLAN_SRC), jnp.asarray(_PLAN_DST))
    return out.reshape(B, T, D)


# speedup vs baseline: 1.2151x; 1.2081x over previous
import functools

import numpy as np
import jax
import jax.numpy as jnp
from jax import lax
from jax.experimental import pallas as pl
from jax.experimental.pallas import tpu as pltpu, tpu_sc as plsc

B, T, D = 128, 4096, 64
NC, NS = 2, 16
NW = NC * NS
C = 256
RW = C + 8
AMAX = T - RW


def _build_plan():
    rng = np.random.RandomState(0)
    orig = np.arange(T)
    num_segs = rng.randint(1, 5, size=B)
    warp = np.tile(orig, (B, 1))
    for i in range(B):
        if num_segs[i] > 1:
            splits = np.array_split(orig, num_segs[i])
            perm = rng.permutation(len(splits))
            warp[i] = np.concatenate([splits[j] for j in perm]).ravel()

    plain = []
    strad = {}
    for i in range(B):
        row = warp[i]
        breaks = np.flatnonzero(np.diff(row) != 1)
        dst_starts = np.concatenate(([0], breaks + 1))
        dst_ends = np.concatenate((breaks + 1, [T]))
        srcs = row[dst_starts]
        for c in range(T // C):
            lo, hi = c * C, (c + 1) * C
            r = int(np.searchsorted(dst_ends, lo, side="right"))
            s1 = int(srcs[r] + (lo - dst_starts[r]))
            if dst_ends[r] >= hi:
                plain.append((i, c, s1))
            else:
                l1 = int(dst_ends[r] - lo)
                s2 = int(srcs[r + 1])
                assert dst_ends[r + 1] >= hi
                strad.setdefault(c, []).append((i, s1, l1, s2))

    while len(plain) % NW:
        plain.append(plain[0])
    for c in strad:
        while len(strad[c]) % NW:
            strad[c].append(strad[c][0])

    na = len(plain) // NW
    ngrp = -(-na // 8)
    ncol = ngrp * 8 + 8
    pa = np.zeros((NW, ncol), dtype=np.int32)
    for k, (i, c, s1) in enumerate(plain):
        pa[k % NW, k // NW] = (i << 17) | (c << 13) | s1

    classes = []
    pbc = np.zeros((NW, 16), dtype=np.int32)
    col = 0
    for c in sorted(strad):
        lvals = [l1 for (_, _, l1, _) in strad[c]]
        w1 = min(lvals) & ~7
        assert max(lvals) < w1 + 8
        n_cls = len(strad[c]) // NW
        classes.append((c * C, w1, n_cls))
        for k, (i, s1, l1, s2) in enumerate(strad[c]):
            pbc[k % NW, col + 2 * (k // NW)] = (i << 13) | s1
            pbc[k % NW, col + 2 * (k // NW) + 1] = (l1 << 13) | s2
        col += 2 * n_cls
    assert col <= 16
    return pa, pbc, na, ngrp, ncol, classes


_PLAN_A, _PLAN_BC, NA, NGRP, NCOL, _CLASSES = _build_plan()


@functools.partial(
    pl.kernel,
    out_type=jax.ShapeDtypeStruct((B, T, D), jnp.float32),
    mesh=plsc.VectorSubcoreMesh(core_axis_name="c", subcore_axis_name="s"),
    scratch_types=[
        pltpu.VMEM((NCOL,), jnp.int32),
        pltpu.VMEM((16,), jnp.int32),
        pltpu.VMEM((RW, D), jnp.float32),
        pltpu.VMEM((RW, D), jnp.float32),
        pltpu.VMEM((8, D), jnp.float32),
        pltpu.SemaphoreType.DMA,
        pltpu.SemaphoreType.DMA,
        pltpu.SemaphoreType.DMA,
        pltpu.SemaphoreType.DMA,
    ],
)
def _sc_permute(x_hbm, pa_hbm, pbc_hbm, out_hbm,
                pa_v, pbc_v, rbuf0, rbuf1, mbuf,
                rsem0, rsem1, wsem0, wsem1):
    wid = lax.axis_index("s") * NC + lax.axis_index("c")
    pltpu.sync_copy(pa_hbm.at[wid], pa_v)
    pltpu.sync_copy(pbc_hbm.at[wid], pbc_v)
    bufs = (rbuf0, rbuf1)
    rsems = (rsem0, rsem1)
    wsems = (wsem0, wsem1)

    def body(g, carry):
        base = pl.multiple_of(g * 8, 8)
        vec = pa_v[pl.ds(base, 16)]
        for k in range(8):
            jj = g * 8 + k
            buf, rsem, wsem = bufs[k & 1], rsems[k & 1], wsems[k & 1]

            @pl.when((jj >= 2) & (jj < NA))
            def _():
                pltpu.make_async_copy(
                    buf.at[pl.ds(0, C)], out_hbm.at[0, pl.ds(0, C)],
                    wsem).wait()

            @pl.when(jj < NA)
            def _():
                p = vec[k]
                i = p >> 17
                c = (p >> 13) & 15
                s1 = p & 8191
                a1 = pl.multiple_of(jnp.minimum(s1 & ~7, AMAX), 8)
                r1 = s1 - a1
                d = pl.multiple_of(c * C, 8)
                pltpu.async_copy(x_hbm.at[i, pl.ds(a1, RW)], buf,
                                 rsem).wait()
                pltpu.async_copy(buf.at[pl.ds(r1, C)],
                                 out_hbm.at[i, pl.ds(d, C)], wsem)

        return carry

    lax.fori_loop(0, NGRP, body, 0)
    pltpu.make_async_copy(rbuf0.at[pl.ds(0, C)],
                          out_hbm.at[0, pl.ds(0, C)], wsem0).wait()
    pltpu.make_async_copy(rbuf1.at[pl.ds(0, C)],
                          out_hbm.at[0, pl.ds(0, C)], wsem1).wait()

    bc = pbc_v[pl.ds(0, 16)]
    col = 0
    for dst0, w1, n_cls in _CLASSES:
        w2 = C - w1 - 8
        for jb in range(n_cls):
            p1 = bc[col + 2 * jb]
            p2 = bc[col + 2 * jb + 1]
            i = p1 >> 13
            s1 = p1 & 8191
            l1 = p2 >> 13
            s2 = p2 & 8191
            a1 = pl.multiple_of(jnp.minimum(s1 & ~7, AMAX), 8)
            r1 = s1 - a1
            a2 = pl.multiple_of(jnp.minimum(s2 & ~7, AMAX), 8)
            r2 = s2 - a2
            pltpu.sync_copy(x_hbm.at[i, pl.ds(a1, RW)], rbuf0)
            pltpu.sync_copy(x_hbm.at[i, pl.ds(a2, RW)], rbuf1)
            pltpu.sync_copy(rbuf0.at[pl.ds(r1, w1)],
                            out_hbm.at[i, pl.ds(dst0, w1)])
            nrow1 = l1 - w1
            for q in range(8):
                for t in range(4):
                    @pl.when(q < nrow1)
                    def _():
                        mbuf[q, pl.ds(t * 16, 16)] = (
                            rbuf0[r1 + w1 + q, pl.ds(t * 16, 16)])

                    @pl.when(q >= nrow1)
                    def _():
                        mbuf[q, pl.ds(t * 16, 16)] = (
                            rbuf1[r2 + q - nrow1, pl.ds(t * 16, 16)])
            pltpu.sync_copy(mbuf, out_hbm.at[i, pl.ds(dst0 + w1, 8)])
            off2 = r2 + (w1 + 8 - l1)
            pltpu.sync_copy(rbuf1.at[pl.ds(off2, w2)],
                            out_hbm.at[i, pl.ds(dst0 + w1 + 8, w2)])
        col += 2 * n_cls


def kernel(x):
    return _sc_permute(x, jnp.asarray(_PLAN_A), jnp.asarray(_PLAN_BC))


# speedup vs baseline: 3.7124x; 3.0553x over previous
import functools

import numpy as np
import jax
import jax.numpy as jnp
from jax import lax
from jax.experimental import pallas as pl
from jax.experimental.pallas import tpu as pltpu, tpu_sc as plsc

B, T, D = 128, 4096, 64
NC, NS = 2, 16
NW = NC * NS
C = 256
W = C + 128
AMAX = T - W
NV = C // 16


def _build_plan():
    rng = np.random.RandomState(0)
    orig = np.arange(T)
    num_segs = rng.randint(1, 5, size=B)
    warp = np.tile(orig, (B, 1))
    for i in range(B):
        if num_segs[i] > 1:
            splits = np.array_split(orig, num_segs[i])
            perm = rng.permutation(len(splits))
            warp[i] = np.concatenate([splits[j] for j in perm]).ravel()

    fast = []
    shift = []
    strad = []
    for i in range(B):
        row = warp[i]
        breaks = np.flatnonzero(np.diff(row) != 1)
        dst_starts = np.concatenate(([0], breaks + 1))
        dst_ends = np.concatenate((breaks + 1, [T]))
        srcs = row[dst_starts]
        for c in range(T // C):
            lo, hi = c * C, (c + 1) * C
            r = int(np.searchsorted(dst_ends, lo, side="right"))
            s1 = int(srcs[r] + (lo - dst_starts[r]))
            if dst_ends[r] >= hi:
                (fast if s1 % 128 == 0 else shift).append((i, c, s1))
            else:
                l1 = int(dst_ends[r] - lo)
                s2 = int(srcs[r + 1])
                assert dst_ends[r + 1] >= hi
                strad.append((i, c, s1, l1, s2))

    for lst in (fast, shift, strad):
        while lst and len(lst) % NW:
            lst.append(lst[0])

    na = len(fast) // NW
    ngrp = -(-na // 8)
    ncol = ngrp * 8 + 8
    pa = np.zeros((NW, ncol), dtype=np.int32)
    for k, (i, c, s1) in enumerate(fast):
        pa[k % NW, k // NW] = (i << 17) | (c << 13) | s1

    nsh = len(shift) // NW
    nst = len(strad) // NW
    assert nsh + nst <= 16, (nsh, nst)
    pbc = np.zeros((NW, 32), dtype=np.int32)
    for k, (i, c, s1) in enumerate(shift):
        pbc[k % NW, k // NW] = (i << 13) | s1
        pbc[k % NW, 16 + k // NW] = (c << 22) | (C << 13)
    for k, (i, c, s1, l1, s2) in enumerate(strad):
        pbc[k % NW, nsh + k // NW] = (i << 13) | s1
        pbc[k % NW, 16 + nsh + k // NW] = (c << 22) | (l1 << 13) | s2
    return pa, pbc, na, ngrp, ncol, nsh, nst


_PLAN_A, _PLAN_BC, NA, NGRP, NCOL, NSH, NST = _build_plan()


@functools.partial(
    pl.kernel,
    out_type=jax.ShapeDtypeStruct((B, D, T), jnp.float32),
    mesh=plsc.VectorSubcoreMesh(core_axis_name="c", subcore_axis_name="s"),
    scratch_types=[
        pltpu.VMEM((NCOL,), jnp.int32),
        pltpu.VMEM((32,), jnp.int32),
        pltpu.VMEM((D, W), jnp.float32),
        pltpu.VMEM((D, W), jnp.float32),
        pltpu.SemaphoreType.DMA,
        pltpu.SemaphoreType.DMA,
        pltpu.SemaphoreType.DMA,
        pltpu.SemaphoreType.DMA,
    ],
    compiler_params=pltpu.CompilerParams(needs_layout_passes=False),
)
def _sc_permute(x_hbm, pa_hbm, pbc_hbm, out_hbm,
                pa_v, pbc_v, rbuf0, rbuf1,
                rsem0, rsem1, wsem0, wsem1):
    wid = lax.axis_index("s") * NC + lax.axis_index("c")
    pltpu.sync_copy(pa_hbm.at[wid], pa_v)
    pltpu.sync_copy(pbc_hbm.at[wid], pbc_v)
    bufs = (rbuf0, rbuf1)
    rsems = (rsem0, rsem1)
    wsems = (wsem0, wsem1)

    def body(g, carry):
        base = pl.multiple_of(g * 8, 8)
        vec = pa_v[pl.ds(base, 16)]
        for k in range(8):
            jj = g * 8 + k
            buf, rsem, wsem = bufs[k & 1], rsems[k & 1], wsems[k & 1]

            @pl.when((jj >= 2) & (jj < NA))
            def _():
                pltpu.make_async_copy(
                    buf.at[:, pl.ds(0, C)], out_hbm.at[0, :, pl.ds(0, C)],
                    wsem).wait()

            @pl.when(jj < NA)
            def _():
                p = vec[k]
                i = p >> 17
                c = (p >> 13) & 15
                s1 = pl.multiple_of(p & 8191, 128)
                d = pl.multiple_of(c * C, 128)
                pltpu.async_copy(x_hbm.at[i, :, pl.ds(s1, C)],
                                 buf.at[:, pl.ds(0, C)], rsem).wait()
                pltpu.async_copy(buf.at[:, pl.ds(0, C)],
                                 out_hbm.at[i, :, pl.ds(d, C)], wsem)

        return carry

    lax.fori_loop(0, NGRP, body, 0)
    pltpu.make_async_copy(rbuf0.at[:, pl.ds(0, C)],
                          out_hbm.at[0, :, pl.ds(0, C)], wsem0).wait()
    pltpu.make_async_copy(rbuf1.at[:, pl.ds(0, C)],
                          out_hbm.at[0, :, pl.ds(0, C)], wsem1).wait()

    bc1 = pbc_v[pl.ds(0, 16)]
    bc2 = pbc_v[pl.ds(16, 16)]
    tvs = [jnp.arange(16, dtype=jnp.int32) + 16 * j for j in range(NV)]

    for jb in range(NSH + NST):
        is_strad = jb >= NSH
        p1 = bc1[jb]
        p2 = bc2[jb]
        i = p1 >> 13
        s1 = p1 & 8191
        c = p2 >> 22
        l1 = (p2 >> 13) & 511
        s2 = p2 & 8191
        a1 = pl.multiple_of(jnp.minimum(s1 & ~127, AMAX), 128)
        r1 = s1 - a1
        pltpu.sync_copy(x_hbm.at[i, :, pl.ds(a1, W)], rbuf0)
        if is_strad:
            a2 = pl.multiple_of(jnp.minimum(s2 & ~127, AMAX), 128)
            r2 = s2 - a2
            pltpu.sync_copy(x_hbm.at[i, :, pl.ds(a2, W)], rbuf1)

        def frow(f, carry):
            fv = jnp.full((16,), 0, jnp.int32) + f
            for j in range(NV):
                col1 = r1 + tvs[j]
                if is_strad:
                    col1 = jnp.minimum(col1, W - 1)
                g1 = plsc.load_gather(rbuf0, [fv, col1])
                if is_strad:
                    col2 = jnp.maximum(r2 + tvs[j] - l1, 0)
                    g2 = plsc.load_gather(rbuf1, [fv, col2])
                    g1 = jnp.where(tvs[j] < l1, g1, g2)
                rbuf0[f, pl.ds(16 * j, 16)] = g1
            return carry

        lax.fori_loop(0, D, frow, 0)
        d = pl.multiple_of(c * C, 128)
        pltpu.sync_copy(rbuf0.at[:, pl.ds(0, C)],
                        out_hbm.at[i, :, pl.ds(d, C)])


def kernel(x):
    xt = jnp.transpose(x, (0, 2, 1))
    out = _sc_permute(xt, jnp.asarray(_PLAN_A), jnp.asarray(_PLAN_BC))
    return jnp.transpose(out, (0, 2, 1))
